# Initial kernel scaffold; baseline (speedup 1.0000x reference)
import jax
import jax.numpy as jnp
from jax.experimental import pallas as pl


def kernel(x, table):
    raise NotImplementedError("write your pallas kernel here")


# speedup vs baseline: 4.1446x; 4.1446x over previous
import functools

import jax
import jax.numpy as jnp
from jax import lax
from jax.experimental import pallas as pl
from jax.experimental.pallas import tpu as pltpu
from jax.experimental.pallas import tpu_sc as plsc

D = 32
T = 7
NW = 32
LANES = 16
IW = 128


@functools.lru_cache(maxsize=None)
def _build(N, V, C, interpret=False):
    rows_per_w = N // NW
    n_chunks = rows_per_w // C
    idx_per_chunk = C * T
    g_per_chunk = idx_per_chunk // IW

    mesh = plsc.VectorSubcoreMesh(core_axis_name="c", subcore_axis_name="s")

    @functools.partial(
        pl.kernel,
        out_type=jax.ShapeDtypeStruct((N, D), jnp.float32),
        mesh=mesh,
        scratch_types=[
            pltpu.VMEM((idx_per_chunk,), jnp.int32),
            pltpu.VMEM((C * T, D), jnp.float32),
            pltpu.VMEM((C, D), jnp.float32),
            pltpu.SemaphoreType.DMA,
        ],
        compiler_params=pltpu.CompilerParams(use_tc_tiling_on_sc=False),
        interpret=interpret,
    )
    def k(x_hbm, table_hbm, out_hbm, idx_v, gbuf, obuf, sem):
        wid = lax.axis_index("s") * 2 + lax.axis_index("c")

        def chunk(ch, _):
            xoff0 = wid * (rows_per_w * T) + ch * idx_per_chunk
            orow0 = wid * rows_per_w + ch * C
            pltpu.sync_copy(x_hbm.at[pl.ds(xoff0, idx_per_chunk)], idx_v)
            copies = []
            for j in range(g_per_chunk):
                copies.append(
                    pltpu.async_copy(
                        table_hbm.at[idx_v.at[pl.ds(j * IW, IW)]],
                        gbuf.at[pl.ds(j * IW, IW)],
                        sem,
                    )
                )
            for c in copies:
                c.wait()

            def row(r, _):
                base = r * T
                for h in range(D // LANES):
                    acc = gbuf[base, pl.ds(h * LANES, LANES)]
                    for t in range(1, T):
                        acc = acc + gbuf[base + t, pl.ds(h * LANES, LANES)]
                    obuf[r, pl.ds(h * LANES, LANES)] = acc
                return 0

            lax.fori_loop(0, C, row, 0, unroll=4)
            pltpu.sync_copy(obuf, out_hbm.at[pl.ds(orow0, C)])
            return 0

        lax.fori_loop(0, n_chunks, chunk, 0)

    return k


def kernel(x, table):
    B, L, t = x.shape
    N = B * L
    V, d = table.shape
    x1d = x.astype(jnp.int32).reshape(N * T)
    rows_per_w = N // NW
    C = 128 if rows_per_w % 128 == 0 else rows_per_w
    out = _build(N, V, C)(x1d, table)
    return out.reshape(B, L, D)


# speedup vs baseline: 4.3802x; 1.0568x over previous
import functools

import jax
import jax.numpy as jnp
from jax import lax
from jax.experimental import pallas as pl
from jax.experimental.pallas import tpu as pltpu
from jax.experimental.pallas import tpu_sc as plsc

D = 32
T = 7
NW = 32
IW = 128


@functools.lru_cache(maxsize=None)
def _build(N, V, C):
    rows_per_w = N // NW
    n_chunks = rows_per_w // C
    K = C // IW
    jrows = N // IW
    NBUF = 2

    mesh = plsc.VectorSubcoreMesh(core_axis_name="c", subcore_axis_name="s")

    @functools.partial(
        pl.kernel,
        out_type=jax.ShapeDtypeStruct((N, D), jnp.float32),
        mesh=mesh,
        scratch_types=[
            pltpu.VMEM((NBUF, T, K, IW), jnp.int32),
            pltpu.VMEM((NBUF, C, D), jnp.float32),
            pltpu.SemaphoreType.DMA((NBUF,)),
            pltpu.SemaphoreType.DMA((NBUF,)),
        ],
        compiler_params=pltpu.CompilerParams(use_tc_tiling_on_sc=False),
    )
    def k(x_hbm, table_hbm, out_hbm, idx_v, acc_v, gsem, wsem):
        wid = lax.axis_index("s") * 2 + lax.axis_index("c")

        def stage(ch, slot):
            jbase = wid * (rows_per_w // IW) + ch * K
            pltpu.sync_copy(
                x_hbm.at[:, pl.ds(jbase, K), :], idx_v.at[slot]
            )
            for j in range(K):
                pltpu.async_copy(
                    table_hbm.at[idx_v.at[slot, 0, j]],
                    acc_v.at[slot, pl.ds(j * IW, IW)],
                    gsem.at[slot],
                )

        def addfire(slot):
            for j in range(K):
                pltpu.make_async_copy(
                    table_hbm.at[idx_v.at[slot, 0, j]],
                    acc_v.at[slot, pl.ds(j * IW, IW)],
                    gsem.at[slot],
                ).wait()
            for t in range(1, T):
                for j in range(K):
                    pltpu.async_copy(
                        table_hbm.at[idx_v.at[slot, t, j]],
                        acc_v.at[slot, pl.ds(j * IW, IW)],
                        gsem.at[slot],
                        add=True,
                    )

        def drain_adds(slot):
            for t in range(1, T):
                for j in range(K):
                    pltpu.make_async_copy(
                        table_hbm.at[idx_v.at[slot, t, j]],
                        acc_v.at[slot, pl.ds(j * IW, IW)],
                        gsem.at[slot],
                    ).wait()

        def writeback(ch, slot):
            orow0 = wid * rows_per_w + ch * C
            pltpu.async_copy(
                acc_v.at[slot], out_hbm.at[pl.ds(orow0, C)], wsem.at[slot]
            )

        def wait_writeback(ch, slot):
            orow0 = wid * rows_per_w + ch * C
            pltpu.make_async_copy(
                acc_v.at[slot], out_hbm.at[pl.ds(orow0, C)], wsem.at[slot]
            ).wait()

        stage(0, 0)
        addfire(0)

        def body(ch, _):
            slot = lax.rem(ch, NBUF)
            nslot = lax.rem(ch + 1, NBUF)

            @pl.when(ch + 1 < n_chunks)
            def _():
                @pl.when(ch + 1 >= NBUF)
                def _():
                    wait_writeback(ch + 1 - NBUF, nslot)
                stage(ch + 1, nslot)
                addfire(nslot)

            drain_adds(slot)
            writeback(ch, slot)
            return 0

        lax.fori_loop(0, n_chunks, body, 0)
        for i in range(NBUF):
            ch = n_chunks - NBUF + i
            wait_writeback(ch, ch % NBUF)

    return k


def kernel(x, table):
    B, L, t = x.shape
    N = B * L
    V, d = table.shape
    xt = x.astype(jnp.int32).reshape(N, T).T.reshape(T, N // IW, IW)
    C = 1024
    out = _build(N, V, C)(xt, table)
    return out.reshape(B, L, D)


# speedup vs baseline: 5.5349x; 1.2636x over previous
import functools

import jax
import jax.numpy as jnp
from jax import lax
from jax.experimental import pallas as pl
from jax.experimental.pallas import tpu as pltpu
from jax.experimental.pallas import tpu_sc as plsc

D = 32
T = 7
NW = 32
IW = 128
LANES = 16


@functools.lru_cache(maxsize=None)
def _build(N, V, C):
    rows_per_w = N // NW
    n_chunks = rows_per_w // C
    K = C // IW
    CT = C * T
    G = C // LANES
    NBUF = 2

    mesh = plsc.VectorSubcoreMesh(core_axis_name="c", subcore_axis_name="s")

    @functools.partial(
        pl.kernel,
        out_type=jax.ShapeDtypeStruct((N, D), jnp.float32),
        mesh=mesh,
        scratch_types=[
            pltpu.VMEM((NBUF * CT,), jnp.int32),
            pltpu.VMEM((NBUF * CT,), jnp.int32),
            pltpu.VMEM((NBUF, C, D), jnp.float32),
            pltpu.SemaphoreType.DMA((NBUF,)),
            pltpu.SemaphoreType.DMA((NBUF,)),
        ],
        compiler_params=pltpu.CompilerParams(
            use_tc_tiling_on_sc=False, needs_layout_passes=False
        ),
    )
    def k(x_hbm, table_hbm, out_hbm, ibuf, tbuf, acc_v, gsem, wsem):
        wid = lax.axis_index("s") * 2 + lax.axis_index("c")

        def stage(ch, slot):
            xoff = wid * (rows_per_w * T) + ch * CT
            ioff = slot * CT
            pltpu.sync_copy(
                x_hbm.at[pl.ds(xoff, CT)], ibuf.at[pl.ds(ioff, CT)]
            )
            lanes7 = lax.iota(jnp.int32, LANES) * T

            def regroup(g, _):
                src0 = ioff + g * (LANES * T)
                for t in range(T):
                    v = plsc.load_gather(ibuf, [lanes7 + (src0 + t)])
                    tbuf[pl.ds(ioff + t * C + g * LANES, LANES)] = v
                return 0

            lax.fori_loop(0, G, regroup, 0, unroll=2)
            for j in range(K):
                pltpu.async_copy(
                    table_hbm.at[tbuf.at[pl.ds(ioff + j * IW, IW)]],
                    acc_v.at[slot, pl.ds(j * IW, IW)],
                    gsem.at[slot],
                )

        def addfire(slot):
            ioff = slot * CT
            for j in range(K):
                pltpu.make_async_copy(
                    table_hbm.at[tbuf.at[pl.ds(ioff + j * IW, IW)]],
                    acc_v.at[slot, pl.ds(j * IW, IW)],
                    gsem.at[slot],
                ).wait()
            for t in range(1, T):
                for j in range(K):
                    pltpu.async_copy(
                        table_hbm.at[tbuf.at[pl.ds(ioff + t * C + j * IW, IW)]],
                        acc_v.at[slot, pl.ds(j * IW, IW)],
                        gsem.at[slot],
                        add=True,
                    )

        def drain_adds(slot):
            ioff = slot * CT
            for t in range(1, T):
                for j in range(K):
                    pltpu.make_async_copy(
                        table_hbm.at[tbuf.at[pl.ds(ioff + t * C + j * IW, IW)]],
                        acc_v.at[slot, pl.ds(j * IW, IW)],
                        gsem.at[slot],
                    ).wait()

        def writeback(ch, slot):
            orow0 = wid * rows_per_w + ch * C
            pltpu.async_copy(
                acc_v.at[slot], out_hbm.at[pl.ds(orow0, C)], wsem.at[slot]
            )

        def wait_writeback(ch, slot):
            orow0 = wid * rows_per_w + ch * C
            pltpu.make_async_copy(
                acc_v.at[slot], out_hbm.at[pl.ds(orow0, C)], wsem.at[slot]
            ).wait()

        stage(0, 0)
        addfire(0)

        def body(ch, _):
            slot = lax.rem(ch, NBUF)
            nslot = lax.rem(ch + 1, NBUF)

            @pl.when(ch + 1 < n_chunks)
            def _():
                @pl.when(ch + 1 >= NBUF)
                def _():
                    wait_writeback(ch + 1 - NBUF, nslot)
                stage(ch + 1, nslot)
                addfire(nslot)

            drain_adds(slot)
            writeback(ch, slot)
            return 0

        lax.fori_loop(0, n_chunks, body, 0)
        for i in range(NBUF):
            ch = n_chunks - NBUF + i
            wait_writeback(ch, ch % NBUF)

    return k


def kernel(x, table):
    B, L, t = x.shape
    N = B * L
    V, d = table.shape
    x1d = x.astype(jnp.int32).reshape(N * T)
    C = 1024
    out = _build(N, V, C)(x1d, table)
    return out.reshape(B, L, D)


# speedup vs baseline: 10.5566x; 1.9073x over previous
import functools

import jax
import jax.numpy as jnp
from jax import lax
from jax.experimental import pallas as pl
from jax.experimental.pallas import tpu as pltpu
from jax.experimental.pallas import tpu_sc as plsc

D = 32
T = 7
NW = 32
IW = 128
NBUF = 2


@functools.lru_cache(maxsize=None)
def _build(B, L, V):
    CB = B // NW
    K = CB // IW
    CT = CB * T

    mesh = plsc.VectorSubcoreMesh(core_axis_name="c", subcore_axis_name="s")

    @functools.partial(
        pl.kernel,
        out_type=jax.ShapeDtypeStruct((L, B, D), jnp.float32),
        mesh=mesh,
        scratch_types=[
            pltpu.VMEM((NBUF * CT,), jnp.int32),
            pltpu.VMEM((NBUF, CB, D), jnp.float32),
            pltpu.SemaphoreType.DMA((NBUF,)),
            pltpu.SemaphoreType.DMA((NBUF,)),
            pltpu.SemaphoreType.DMA((NBUF,)),
        ],
        compiler_params=pltpu.CompilerParams(
            use_tc_tiling_on_sc=False, needs_layout_passes=False
        ),
    )
    def k(xt_hbm, table_hbm, out_hbm, ibuf, acc_v, ssem, gsem, wsem):
        wid = lax.axis_index("s") * 2 + lax.axis_index("c")
        b0 = wid * CB

        def stage(l, slot):
            ioff = slot * CT
            for t in range(T):
                pltpu.async_copy(
                    xt_hbm.at[t, l, pl.ds(b0, CB)],
                    ibuf.at[pl.ds(ioff + t * CB, CB)],
                    ssem.at[slot],
                )

        def fire0(l, slot):
            ioff = slot * CT
            for t in range(T):
                pltpu.make_async_copy(
                    xt_hbm.at[t, l, pl.ds(b0, CB)],
                    ibuf.at[pl.ds(ioff + t * CB, CB)],
                    ssem.at[slot],
                ).wait()
            for j in range(K):
                pltpu.async_copy(
                    table_hbm.at[ibuf.at[pl.ds(ioff + j * IW, IW)]],
                    acc_v.at[slot, pl.ds(j * IW, IW)],
                    gsem.at[slot],
                )

        def addfire(slot):
            ioff = slot * CT
            for j in range(K):
                pltpu.make_async_copy(
                    table_hbm.at[ibuf.at[pl.ds(ioff + j * IW, IW)]],
                    acc_v.at[slot, pl.ds(j * IW, IW)],
                    gsem.at[slot],
                ).wait()
            for t in range(1, T):
                for j in range(K):
                    pltpu.async_copy(
                        table_hbm.at[ibuf.at[pl.ds(ioff + t * CB + j * IW, IW)]],
                        acc_v.at[slot, pl.ds(j * IW, IW)],
                        gsem.at[slot],
                        add=True,
                    )

        def drain_adds(slot):
            ioff = slot * CT
            for t in range(1, T):
                for j in range(K):
                    pltpu.make_async_copy(
                        table_hbm.at[ibuf.at[pl.ds(ioff + t * CB + j * IW, IW)]],
                        acc_v.at[slot, pl.ds(j * IW, IW)],
                        gsem.at[slot],
                    ).wait()

        def writeback(l, slot):
            pltpu.async_copy(
                acc_v.at[slot], out_hbm.at[l, pl.ds(b0, CB)], wsem.at[slot]
            )

        def wait_writeback(l, slot):
            pltpu.make_async_copy(
                acc_v.at[slot], out_hbm.at[l, pl.ds(b0, CB)], wsem.at[slot]
            ).wait()

        stage(0, 0)
        fire0(0, 0)
        addfire(0)

        def body(l, _):
            slot = lax.rem(l, NBUF)
            nslot = lax.rem(l + 1, NBUF)

            @pl.when(l + 1 < L)
            def _():
                @pl.when(l + 1 >= NBUF)
                def _():
                    wait_writeback(l + 1 - NBUF, nslot)
                stage(l + 1, nslot)
                fire0(l + 1, nslot)
                addfire(nslot)

            drain_adds(slot)
            writeback(l, slot)
            return 0

        lax.fori_loop(0, L, body, 0)
        for i in range(NBUF):
            l = L - NBUF + i
            wait_writeback(l, l % NBUF)

    return k


def kernel(x, table):
    B, L, t = x.shape
    V, d = table.shape
    xt = jnp.transpose(x.astype(jnp.int32), (2, 1, 0))
    out_lt = _build(B, L, V)(xt, table)
    return jnp.transpose(out_lt, (1, 0, 2))
